# baseline (speedup 1.0000x reference)
import functools

import jax
import jax.numpy as jnp
from jax import lax
from jax.experimental import pallas as pl
from jax.experimental.pallas import tpu as pltpu
from jax.experimental.pallas import tpu_sc as plsc

_VOCAB = 100000
_D = 32
_NW = 32
_VB = 2048
_NSLOT = 6
_NFULL = _VOCAB // _VB
_LAST = _VOCAB - _NFULL * _VB
_NB = _NFULL + 1


def _sc_gather(flat_t, ids):
    b = ids.shape[0]
    b_per_w = b // _NW
    mesh = plsc.VectorSubcoreMesh(core_axis_name="c", subcore_axis_name="s")

    @functools.partial(
        pl.kernel,
        out_type=jax.ShapeDtypeStruct((_D, b), jnp.float32),
        mesh=mesh,
        scratch_types=[
            pltpu.VMEM((b_per_w,), jnp.int32),
            pltpu.VMEM((_D, b_per_w), jnp.int32),
            pltpu.VMEM((_D, b_per_w), jnp.float32),
            pltpu.SemaphoreType.DMA,
        ],
        compiler_params=pltpu.CompilerParams(use_tc_tiling_on_sc=False),
    )
    def gather_kernel(flat_hbm, ids_hbm, out_hbm, ids_v, idx_v, rows_v, sem):
        wid = lax.axis_index("s") * 2 + lax.axis_index("c")
        base = wid * b_per_w
        pltpu.sync_copy(ids_hbm.at[pl.ds(base, b_per_w)], ids_v)
        tok = ids_v[...]
        for d in range(_D):
            idx_v[d, :] = tok + d * _VOCAB
        copies = [
            pltpu.async_copy(flat_hbm.at[idx_v.at[d]], rows_v.at[d], sem)
            for d in range(_D)
        ]
        for c in copies:
            c.wait()
        pltpu.sync_copy(rows_v, out_hbm.at[:, pl.ds(base, b_per_w)])

    return gather_kernel(flat_t, ids)


def _matmul_body(xt_ref, w_ref, out_hbm, obuf, tbuf, sems, tsem):
    j = pl.program_id(0)
    slot = lax.rem(j, _NSLOT)

    @pl.when(j >= _NSLOT)
    def _():
        pltpu.make_async_copy(
            obuf.at[slot],
            out_hbm.at[:, pl.ds((j - _NSLOT) * _VB, _VB)],
            sems.at[slot],
        ).wait()

    xb = xt_ref[...].astype(jnp.bfloat16)

    @pl.when(j < _NFULL)
    def _():
        obuf[slot] = lax.dot_general(
            xb, w_ref[...].astype(jnp.bfloat16),
            (((0,), (0,)), ((), ())),
            preferred_element_type=jnp.float32,
        )
        pltpu.make_async_copy(
            obuf.at[slot],
            out_hbm.at[:, pl.ds(j * _VB, _VB)],
            sems.at[slot],
        ).start()

    @pl.when(j == _NFULL)
    def _():
        tbuf[...] = lax.dot_general(
            xb, w_ref[:, : _LAST].astype(jnp.bfloat16),
            (((0,), (0,)), ((), ())),
            preferred_element_type=jnp.float32,
        )
        pltpu.make_async_copy(
            tbuf,
            out_hbm.at[:, pl.ds(_NFULL * _VB, _LAST)],
            tsem,
        ).start()
        for back in range(1, _NSLOT):
            pltpu.make_async_copy(
                obuf.at[lax.rem(j - back + _NSLOT, _NSLOT)],
                out_hbm.at[:, pl.ds((_NFULL - back) * _VB, _VB)],
                sems.at[lax.rem(j - back + _NSLOT, _NSLOT)],
            ).wait()
        pltpu.make_async_copy(
            tbuf,
            out_hbm.at[:, pl.ds(_NFULL * _VB, _LAST)],
            tsem,
        ).wait()


def _unembed(xt, wt):
    n = xt.shape[1]
    v = wt.shape[1]
    return pl.pallas_call(
        _matmul_body,
        grid=(_NB,),
        in_specs=[
            pl.BlockSpec((_D, n), lambda j: (0, 0)),
            pl.BlockSpec((_D, _VB), lambda j: (0, j)),
        ],
        out_specs=pl.BlockSpec(memory_space=pl.ANY),
        out_shape=jax.ShapeDtypeStruct((n, v), jnp.float32),
        scratch_shapes=[
            pltpu.VMEM((_NSLOT, n, _VB), jnp.float32),
            pltpu.VMEM((n, _LAST), jnp.float32),
            pltpu.SemaphoreType.DMA((_NSLOT,)),
            pltpu.SemaphoreType.DMA,
        ],
    )(xt, wt)


def kernel(token_ids, embed_table, unembed_w):
    b, s = token_ids.shape
    ids = token_ids.reshape(-1).astype(jnp.int32)
    flat_t = embed_table.T.reshape(-1)
    emb_t = _sc_gather(flat_t, ids)
    logits = _unembed(emb_t, unembed_w.T)
    return logits.reshape(b, s, _VOCAB)


# speedup vs baseline: 1.0186x; 1.0186x over previous
import functools

import jax
import jax.numpy as jnp
from jax import lax
from jax.experimental import pallas as pl
from jax.experimental.pallas import tpu as pltpu
from jax.experimental.pallas import tpu_sc as plsc

_VOCAB = 100000
_D = 32
_NW = 32
_VB = 6144
_NSLOT = 4
_NFULL = _VOCAB // _VB
_LAST = _VOCAB - _NFULL * _VB
_NB = _NFULL + 1


def _sc_gather(flat_t, ids):
    b = ids.shape[0]
    b_per_w = b // _NW
    mesh = plsc.VectorSubcoreMesh(core_axis_name="c", subcore_axis_name="s")

    @functools.partial(
        pl.kernel,
        out_type=jax.ShapeDtypeStruct((_D, b), jnp.float32),
        mesh=mesh,
        scratch_types=[
            pltpu.VMEM((b_per_w,), jnp.int32),
            pltpu.VMEM((_D, b_per_w), jnp.int32),
            pltpu.VMEM((_D, b_per_w), jnp.float32),
            pltpu.SemaphoreType.DMA,
        ],
        compiler_params=pltpu.CompilerParams(use_tc_tiling_on_sc=False),
    )
    def gather_kernel(flat_hbm, ids_hbm, out_hbm, ids_v, idx_v, rows_v, sem):
        wid = lax.axis_index("s") * 2 + lax.axis_index("c")
        base = wid * b_per_w
        pltpu.sync_copy(ids_hbm.at[pl.ds(base, b_per_w)], ids_v)
        tok = ids_v[...]
        for d in range(_D):
            idx_v[d, :] = tok + d * _VOCAB
        copies = [
            pltpu.async_copy(flat_hbm.at[idx_v.at[d]], rows_v.at[d], sem)
            for d in range(_D)
        ]
        for c in copies:
            c.wait()
        pltpu.sync_copy(rows_v, out_hbm.at[:, pl.ds(base, b_per_w)])

    return gather_kernel(flat_t, ids)


def _matmul_body(xt_ref, w_ref, out_hbm, obuf, tbuf, sems, tsem):
    j = pl.program_id(0)
    slot = lax.rem(j, _NSLOT)

    @pl.when(j >= _NSLOT)
    def _():
        pltpu.make_async_copy(
            obuf.at[slot],
            out_hbm.at[:, pl.ds((j - _NSLOT) * _VB, _VB)],
            sems.at[slot],
        ).wait()

    xb = xt_ref[...].astype(jnp.bfloat16)

    @pl.when(j < _NFULL)
    def _():
        obuf[slot] = lax.dot_general(
            xb, w_ref[...].astype(jnp.bfloat16),
            (((0,), (0,)), ((), ())),
            preferred_element_type=jnp.float32,
        )
        pltpu.make_async_copy(
            obuf.at[slot],
            out_hbm.at[:, pl.ds(j * _VB, _VB)],
            sems.at[slot],
        ).start()

    @pl.when(j == _NFULL)
    def _():
        tbuf[...] = lax.dot_general(
            xb, w_ref[:, : _LAST].astype(jnp.bfloat16),
            (((0,), (0,)), ((), ())),
            preferred_element_type=jnp.float32,
        )
        pltpu.make_async_copy(
            tbuf,
            out_hbm.at[:, pl.ds(_NFULL * _VB, _LAST)],
            tsem,
        ).start()
        for back in range(1, _NSLOT):
            pltpu.make_async_copy(
                obuf.at[lax.rem(j - back + _NSLOT, _NSLOT)],
                out_hbm.at[:, pl.ds((_NFULL - back) * _VB, _VB)],
                sems.at[lax.rem(j - back + _NSLOT, _NSLOT)],
            ).wait()
        pltpu.make_async_copy(
            tbuf,
            out_hbm.at[:, pl.ds(_NFULL * _VB, _LAST)],
            tsem,
        ).wait()


def _unembed(xt, wt):
    n = xt.shape[1]
    v = wt.shape[1]
    return pl.pallas_call(
        _matmul_body,
        grid=(_NB,),
        in_specs=[
            pl.BlockSpec((_D, n), lambda j: (0, 0)),
            pl.BlockSpec((_D, _VB), lambda j: (0, j)),
        ],
        out_specs=pl.BlockSpec(memory_space=pl.ANY),
        out_shape=jax.ShapeDtypeStruct((n, v), jnp.float32),
        scratch_shapes=[
            pltpu.VMEM((_NSLOT, n, _VB), jnp.float32),
            pltpu.VMEM((n, _LAST), jnp.float32),
            pltpu.SemaphoreType.DMA((_NSLOT,)),
            pltpu.SemaphoreType.DMA,
        ],
    )(xt, wt)


def kernel(token_ids, embed_table, unembed_w):
    b, s = token_ids.shape
    ids = token_ids.reshape(-1).astype(jnp.int32)
    flat_t = embed_table.T.reshape(-1)
    emb_t = _sc_gather(flat_t, ids)
    logits = _unembed(emb_t, unembed_w.T)
    return logits.reshape(b, s, _VOCAB)


# speedup vs baseline: 1.0188x; 1.0002x over previous
import functools

import jax
import jax.numpy as jnp
from jax import lax
from jax.experimental import pallas as pl
from jax.experimental.pallas import tpu as pltpu
from jax.experimental.pallas import tpu_sc as plsc

_VOCAB = 100000
_D = 32
_NW = 32
_VB = 6144
_NSLOT = 4
_NFULL = _VOCAB // _VB
_LAST = _VOCAB - _NFULL * _VB
_NB = _NFULL + 1


def _sc_gather(flat_t, ids):
    b = ids.shape[0]
    tb = b // 4
    db = _D // 8
    mesh = plsc.VectorSubcoreMesh(core_axis_name="c", subcore_axis_name="s")

    @functools.partial(
        pl.kernel,
        out_type=jax.ShapeDtypeStruct((_D, b), jnp.float32),
        mesh=mesh,
        scratch_types=[
            pltpu.VMEM((tb,), jnp.int32),
            pltpu.VMEM((db, tb), jnp.int32),
            pltpu.VMEM((db, tb), jnp.float32),
            pltpu.SemaphoreType.DMA,
        ],
        compiler_params=pltpu.CompilerParams(use_tc_tiling_on_sc=False),
    )
    def gather_kernel(flat_hbm, ids_hbm, out_hbm, ids_v, idx_v, rows_v, sem):
        wid = lax.axis_index("s") * 2 + lax.axis_index("c")
        a = wid // 8
        d0 = (wid % 8) * db
        pltpu.sync_copy(ids_hbm.at[pl.ds(a * tb, tb)], ids_v)
        for c in range(tb // 16):
            tok = ids_v[pl.ds(16 * c, 16)]
            for r in range(db):
                idx_v[r, pl.ds(16 * c, 16)] = tok + (d0 + r) * _VOCAB
        copies = [
            pltpu.async_copy(flat_hbm.at[idx_v.at[r]], rows_v.at[r], sem)
            for r in range(db)
        ]
        for cp in copies:
            cp.wait()
        pltpu.sync_copy(rows_v, out_hbm.at[pl.ds(d0, db), pl.ds(a * tb, tb)])

    return gather_kernel(flat_t, ids)


def _matmul_body(xt_ref, w_ref, out_hbm, obuf, tbuf, sems, tsem):
    j = pl.program_id(0)
    slot = lax.rem(j, _NSLOT)

    @pl.when(j >= _NSLOT)
    def _():
        pltpu.make_async_copy(
            obuf.at[slot],
            out_hbm.at[:, pl.ds((j - _NSLOT) * _VB, _VB)],
            sems.at[slot],
        ).wait()

    xb = xt_ref[...].astype(jnp.bfloat16)

    @pl.when(j < _NFULL)
    def _():
        obuf[slot] = lax.dot_general(
            xb, w_ref[...].astype(jnp.bfloat16),
            (((0,), (0,)), ((), ())),
            preferred_element_type=jnp.float32,
        )
        pltpu.make_async_copy(
            obuf.at[slot],
            out_hbm.at[:, pl.ds(j * _VB, _VB)],
            sems.at[slot],
        ).start()

    @pl.when(j == _NFULL)
    def _():
        tbuf[...] = lax.dot_general(
            xb, w_ref[:, : _LAST].astype(jnp.bfloat16),
            (((0,), (0,)), ((), ())),
            preferred_element_type=jnp.float32,
        )
        pltpu.make_async_copy(
            tbuf,
            out_hbm.at[:, pl.ds(_NFULL * _VB, _LAST)],
            tsem,
        ).start()
        for back in range(1, _NSLOT):
            pltpu.make_async_copy(
                obuf.at[lax.rem(j - back + _NSLOT, _NSLOT)],
                out_hbm.at[:, pl.ds((_NFULL - back) * _VB, _VB)],
                sems.at[lax.rem(j - back + _NSLOT, _NSLOT)],
            ).wait()
        pltpu.make_async_copy(
            tbuf,
            out_hbm.at[:, pl.ds(_NFULL * _VB, _LAST)],
            tsem,
        ).wait()


def _unembed(xt, wt):
    n = xt.shape[1]
    v = wt.shape[1]
    return pl.pallas_call(
        _matmul_body,
        grid=(_NB,),
        in_specs=[
            pl.BlockSpec((_D, n), lambda j: (0, 0)),
            pl.BlockSpec((_D, _VB), lambda j: (0, j)),
        ],
        out_specs=pl.BlockSpec(memory_space=pl.ANY),
        out_shape=jax.ShapeDtypeStruct((n, v), jnp.float32),
        scratch_shapes=[
            pltpu.VMEM((_NSLOT, n, _VB), jnp.float32),
            pltpu.VMEM((n, _LAST), jnp.float32),
            pltpu.SemaphoreType.DMA((_NSLOT,)),
            pltpu.SemaphoreType.DMA,
        ],
    )(xt, wt)


def kernel(token_ids, embed_table, unembed_w):
    b, s = token_ids.shape
    ids = token_ids.reshape(-1).astype(jnp.int32)
    flat_t = embed_table.T.reshape(-1)
    emb_t = _sc_gather(flat_t, ids)
    logits = _unembed(emb_t, unembed_w.T)
    return logits.reshape(b, s, _VOCAB)


# speedup vs baseline: 1.0226x; 1.0037x over previous
import functools

import jax
import jax.numpy as jnp
from jax import lax
from jax.experimental import pallas as pl
from jax.experimental.pallas import tpu as pltpu
from jax.experimental.pallas import tpu_sc as plsc

_VOCAB = 100000
_D = 32
_NW = 32
_VB = 6144
_NSLOT = 4
_NFULL = _VOCAB // _VB
_LAST = _VOCAB - _NFULL * _VB
_NB = _NFULL + 1


def _sc_gather(flat_t, ids):
    b = ids.shape[0]
    tb = b // 4
    db = _D // 8
    mesh = plsc.VectorSubcoreMesh(core_axis_name="c", subcore_axis_name="s")

    @functools.partial(
        pl.kernel,
        out_type=jax.ShapeDtypeStruct((_D, b), jnp.float32),
        mesh=mesh,
        scratch_types=[
            pltpu.VMEM((tb,), jnp.int32),
            pltpu.VMEM((db, tb), jnp.int32),
            pltpu.VMEM((db, tb), jnp.float32),
            pltpu.SemaphoreType.DMA,
        ],
        compiler_params=pltpu.CompilerParams(use_tc_tiling_on_sc=False),
    )
    def gather_kernel(flat_hbm, ids_hbm, out_hbm, ids_v, idx_v, rows_v, sem):
        wid = lax.axis_index("s") * 2 + lax.axis_index("c")
        a = wid // 8
        d0 = (wid % 8) * db
        pltpu.sync_copy(ids_hbm.at[pl.ds(a * tb, tb)], ids_v)
        for c in range(tb // 16):
            tok = ids_v[pl.ds(16 * c, 16)]
            for r in range(db):
                idx_v[r, pl.ds(16 * c, 16)] = tok + (d0 + r) * _VOCAB
        copies = [
            pltpu.async_copy(flat_hbm.at[idx_v.at[r]], rows_v.at[r], sem)
            for r in range(db)
        ]
        for cp in copies:
            cp.wait()
        pltpu.sync_copy(rows_v, out_hbm.at[pl.ds(d0, db), pl.ds(a * tb, tb)])

    return gather_kernel(flat_t, ids)


def _matmul_body(xt_ref, w_ref, out_hbm, obuf, tbuf, sems, tsem):
    j = pl.program_id(0)
    slot = lax.rem(j, _NSLOT)

    @pl.when(j >= _NSLOT)
    def _():
        for h in range(2):
            pltpu.make_async_copy(
                obuf.at[slot, pl.ds(h * 256, 256)],
                out_hbm.at[pl.ds(h * 256, 256), pl.ds((j - _NSLOT) * _VB, _VB)],
                sems.at[slot, h],
            ).wait()

    xb = xt_ref[...].astype(jnp.bfloat16)

    @pl.when(j < _NFULL)
    def _():
        obuf[slot] = lax.dot_general(
            xb, w_ref[...].astype(jnp.bfloat16),
            (((0,), (0,)), ((), ())),
            preferred_element_type=jnp.float32,
        )
        for h in range(2):
            pltpu.make_async_copy(
                obuf.at[slot, pl.ds(h * 256, 256)],
                out_hbm.at[pl.ds(h * 256, 256), pl.ds(j * _VB, _VB)],
                sems.at[slot, h],
            ).start()

    @pl.when(j == _NFULL)
    def _():
        tbuf[...] = lax.dot_general(
            xb, w_ref[:, : _LAST].astype(jnp.bfloat16),
            (((0,), (0,)), ((), ())),
            preferred_element_type=jnp.float32,
        )
        pltpu.make_async_copy(
            tbuf,
            out_hbm.at[:, pl.ds(_NFULL * _VB, _LAST)],
            tsem,
        ).start()
        for back in range(1, _NSLOT):
            for h in range(2):
                pltpu.make_async_copy(
                    obuf.at[lax.rem(j - back + _NSLOT, _NSLOT), pl.ds(h * 256, 256)],
                    out_hbm.at[pl.ds(h * 256, 256), pl.ds((_NFULL - back) * _VB, _VB)],
                    sems.at[lax.rem(j - back + _NSLOT, _NSLOT), h],
                ).wait()
        pltpu.make_async_copy(
            tbuf,
            out_hbm.at[:, pl.ds(_NFULL * _VB, _LAST)],
            tsem,
        ).wait()


def _unembed(xt, wt):
    n = xt.shape[1]
    v = wt.shape[1]
    return pl.pallas_call(
        _matmul_body,
        grid=(_NB,),
        in_specs=[
            pl.BlockSpec((_D, n), lambda j: (0, 0)),
            pl.BlockSpec((_D, _VB), lambda j: (0, j)),
        ],
        out_specs=pl.BlockSpec(memory_space=pl.ANY),
        out_shape=jax.ShapeDtypeStruct((n, v), jnp.float32),
        scratch_shapes=[
            pltpu.VMEM((_NSLOT, n, _VB), jnp.float32),
            pltpu.VMEM((n, _LAST), jnp.float32),
            pltpu.SemaphoreType.DMA((_NSLOT, 2)),
            pltpu.SemaphoreType.DMA,
        ],
    )(xt, wt)


def kernel(token_ids, embed_table, unembed_w):
    b, s = token_ids.shape
    ids = token_ids.reshape(-1).astype(jnp.int32)
    flat_t = embed_table.T.reshape(-1)
    emb_t = _sc_gather(flat_t, ids)
    logits = _unembed(emb_t, unembed_w.T)
    return logits.reshape(b, s, _VOCAB)


# speedup vs baseline: 1.0280x; 1.0053x over previous
import functools

import jax
import jax.numpy as jnp
from jax import lax
from jax.experimental import pallas as pl
from jax.experimental.pallas import tpu as pltpu
from jax.experimental.pallas import tpu_sc as plsc

_VOCAB = 100000
_D = 32
_NW = 32
_VB = 6144
_NSLOT = 4
_NFULL = _VOCAB // _VB
_LAST = _VOCAB - _NFULL * _VB
_NB = _NFULL + 1


def _sc_gather(flat_t, ids):
    b = ids.shape[0]
    tb = b // 4
    db = _D // 8
    mesh = plsc.VectorSubcoreMesh(core_axis_name="c", subcore_axis_name="s")

    @functools.partial(
        pl.kernel,
        out_type=jax.ShapeDtypeStruct((_D, b), jnp.float32),
        mesh=mesh,
        scratch_types=[
            pltpu.VMEM((tb,), jnp.int32),
            pltpu.VMEM((db, tb), jnp.int32),
            pltpu.VMEM((db, tb), jnp.float32),
            pltpu.SemaphoreType.DMA,
        ],
        compiler_params=pltpu.CompilerParams(use_tc_tiling_on_sc=False),
    )
    def gather_kernel(flat_hbm, ids_hbm, out_hbm, ids_v, idx_v, rows_v, sem):
        wid = lax.axis_index("s") * 2 + lax.axis_index("c")
        a = wid // 8
        d0 = (wid % 8) * db
        pltpu.sync_copy(ids_hbm.at[pl.ds(a * tb, tb)], ids_v)
        for c in range(tb // 16):
            tok = ids_v[pl.ds(16 * c, 16)]
            for r in range(db):
                idx_v[r, pl.ds(16 * c, 16)] = tok + (d0 + r) * _VOCAB
        copies = [
            pltpu.async_copy(flat_hbm.at[idx_v.at[r]], rows_v.at[r], sem)
            for r in range(db)
        ]
        for cp in copies:
            cp.wait()
        pltpu.sync_copy(rows_v, out_hbm.at[pl.ds(d0, db), pl.ds(a * tb, tb)])

    return gather_kernel(flat_t, ids)


def _matmul_body(xt_ref, w_ref, out_hbm, obuf, tbuf, sems, tsem):
    j = pl.program_id(0)
    f = j - 1
    slot = lax.rem(f, _NSLOT)

    xb = xt_ref[...].astype(jnp.bfloat16)

    @pl.when(j == 0)
    def _():
        tbuf[...] = lax.dot_general(
            xb, w_ref[:, : _LAST].astype(jnp.bfloat16),
            (((0,), (0,)), ((), ())),
            preferred_element_type=jnp.float32,
        )
        pltpu.make_async_copy(
            tbuf,
            out_hbm.at[:, pl.ds(_NFULL * _VB, _LAST)],
            tsem,
        ).start()

    @pl.when(j >= 1)
    def _():
        @pl.when(f >= _NSLOT)
        def _():
            pltpu.make_async_copy(
                obuf.at[slot],
                out_hbm.at[:, pl.ds((f - _NSLOT) * _VB, _VB)],
                sems.at[slot],
            ).wait()

        obuf[slot] = lax.dot_general(
            xb, w_ref[...].astype(jnp.bfloat16),
            (((0,), (0,)), ((), ())),
            preferred_element_type=jnp.float32,
        )
        pltpu.make_async_copy(
            obuf.at[slot],
            out_hbm.at[:, pl.ds(f * _VB, _VB)],
            sems.at[slot],
        ).start()

    @pl.when(j == _NFULL)
    def _():
        for back in range(1, _NSLOT):
            pltpu.make_async_copy(
                obuf.at[lax.rem(f - back + _NSLOT, _NSLOT)],
                out_hbm.at[:, pl.ds((_NFULL - 1 - back) * _VB, _VB)],
                sems.at[lax.rem(f - back + _NSLOT, _NSLOT)],
            ).wait()
        pltpu.make_async_copy(
            obuf.at[slot],
            out_hbm.at[:, pl.ds(f * _VB, _VB)],
            sems.at[slot],
        ).wait()
        pltpu.make_async_copy(
            tbuf,
            out_hbm.at[:, pl.ds(_NFULL * _VB, _LAST)],
            tsem,
        ).wait()


def _unembed(xt, wt):
    n = xt.shape[1]
    v = wt.shape[1]
    return pl.pallas_call(
        _matmul_body,
        grid=(_NB,),
        in_specs=[
            pl.BlockSpec((_D, n), lambda j: (0, 0)),
            pl.BlockSpec((_D, _VB), lambda j: (0, (j + _NFULL) % _NB)),
        ],
        out_specs=pl.BlockSpec(memory_space=pl.ANY),
        out_shape=jax.ShapeDtypeStruct((n, v), jnp.float32),
        scratch_shapes=[
            pltpu.VMEM((_NSLOT, n, _VB), jnp.float32),
            pltpu.VMEM((n, _LAST), jnp.float32),
            pltpu.SemaphoreType.DMA((_NSLOT,)),
            pltpu.SemaphoreType.DMA,
        ],
    )(xt, wt)


def kernel(token_ids, embed_table, unembed_w):
    b, s = token_ids.shape
    ids = token_ids.reshape(-1).astype(jnp.int32)
    flat_t = embed_table.T.reshape(-1)
    emb_t = _sc_gather(flat_t, ids)
    logits = _unembed(emb_t, unembed_w.T)
    return logits.reshape(b, s, _VOCAB)
